# Initial kernel scaffold; baseline (speedup 1.0000x reference)
import jax
import jax.numpy as jnp
from jax.experimental import pallas as pl


def kernel(vxd_hits, vxd_trackids, vxd_tracks, cdc_hits, cdc_trackids, cdc_tracks, params):
    raise NotImplementedError("write your pallas kernel here")


# speedup vs baseline: 8.0651x; 8.0651x over previous
import functools

import jax
import jax.numpy as jnp
from jax.experimental import pallas as pl
from jax.experimental.pallas import tpu as pltpu

H = 32
NV = 512
NC = 1024
EPS = 1e-5
BV = 64
BC = 128


def _ln(x, g, b):
    mu = jnp.mean(x, axis=-1, keepdims=True)
    var = jnp.mean((x - mu) ** 2, axis=-1, keepdims=True)
    return (x - mu) * jax.lax.rsqrt(var + EPS) * g + b


def _embed_scatter_kernel(hits_ref, ids_ref, w_ref, b_ref, g_ref, beta_ref,
                          acc_ref, *, n_tracks):
    i = pl.program_id(0)
    h = jnp.dot(hits_ref[...], w_ref[...],
                preferred_element_type=jnp.float32) + b_ref[...]
    h = jnp.maximum(_ln(h, g_ref[...], beta_ref[...]), 0.0)
    ids = ids_ref[...]
    onehot = (jax.lax.broadcasted_iota(jnp.int32, (ids.shape[0], n_tracks), 1)
              == ids).astype(jnp.float32)
    contrib = jax.lax.dot_general(
        onehot, h, (((0,), (0,)), ((), ())),
        preferred_element_type=jnp.float32)

    @pl.when(i == 0)
    def _():
        acc_ref[...] = jnp.zeros_like(acc_ref)

    acc_ref[...] += contrib


def _embed_scatter(hits, ids, w, b, g, beta, n_tracks, bh):
    n, f = hits.shape
    grid = n // bh
    ids2 = ids.reshape(n, 1)
    b2, g2, beta2 = b.reshape(1, H), g.reshape(1, H), beta.reshape(1, H)
    return pl.pallas_call(
        functools.partial(_embed_scatter_kernel, n_tracks=n_tracks),
        grid=(grid,),
        in_specs=[
            pl.BlockSpec((bh, f), lambda i: (i, 0)),
            pl.BlockSpec((bh, 1), lambda i: (i, 0)),
            pl.BlockSpec((f, H), lambda i: (0, 0)),
            pl.BlockSpec((1, H), lambda i: (0, 0)),
            pl.BlockSpec((1, H), lambda i: (0, 0)),
            pl.BlockSpec((1, H), lambda i: (0, 0)),
        ],
        out_specs=pl.BlockSpec((n_tracks, H), lambda i: (0, 0)),
        out_shape=jax.ShapeDtypeStruct((n_tracks, H), jnp.float32),
    )(hits, ids2, w, b2, g2, beta2)


def _graph_kernel(sv_ref, sc_ref, tv_ref, tc_ref, wtv_ref, wtc_ref,
                  mats_ref, vecs_ref, eout_ref, a_ref, mi_ref, mo_ref,
                  xv_ref, xc_ref, A_ref, B_ref):
    mats = mats_ref[...]
    vecs = vecs_ref[...]

    def M(k):
        return mats[k]

    def V(k):
        return vecs[k:k + 1]

    def lnl(h, w, b, g, t):
        h = jnp.dot(h, w, preferred_element_type=jnp.float32) + b
        return jnp.maximum(_ln(h, g, t), 0.0)

    xv_ref[...] = sv_ref[...] + jnp.maximum(
        _ln(jnp.dot(tv_ref[...], wtv_ref[...],
                    preferred_element_type=jnp.float32) + V(0), V(1), V(2)),
        0.0)
    xc_ref[...] = sc_ref[...] + jnp.maximum(
        _ln(jnp.dot(tc_ref[...], wtc_ref[...],
                    preferred_element_type=jnp.float32) + V(3), V(4), V(5)),
        0.0)

    def edge_pass(mb, vb_, write_out):
        w1a, w1b, w2, w3 = M(mb), M(mb + 1), M(mb + 2), M(mb + 3)
        b1, g1, t1 = V(vb_), V(vb_ + 1), V(vb_ + 2)
        b2, g2, t2 = V(vb_ + 3), V(vb_ + 4), V(vb_ + 5)
        b3, g3, t3 = V(vb_ + 6), V(vb_ + 7), V(vb_ + 8)
        w4r = V(vb_ + 9)
        b4 = vecs[vb_ + 10, 0]
        A_ref[...] = jnp.dot(xv_ref[...], w1a,
                             preferred_element_type=jnp.float32)
        B_ref[...] = jnp.dot(xc_ref[...], w1b,
                             preferred_element_type=jnp.float32)
        if not write_out:
            mi_ref[...] = jnp.zeros_like(mi_ref)
            mo_ref[...] = jnp.zeros_like(mo_ref)

        def body(k, carry):
            vb = k // (NC // BC)
            cb = k % (NC // BC)
            Ab = A_ref[pl.ds(vb * BV, BV), :]
            Bb = B_ref[pl.ds(cb * BC, BC), :]
            pre = Ab[:, None, :] + Bb[None, :, :] + b1[None]
            h = jnp.maximum(_ln(pre, g1[None], t1[None]), 0.0)
            h = h.reshape(BV * BC, H)
            h = lnl(h, w2, b2, g2, t2)
            h = lnl(h, w3, b3, g3, t3)
            h3 = h.reshape(BV, BC, H)
            e = jax.nn.sigmoid(jnp.sum(h3 * w4r[None], axis=-1) + b4)
            if write_out:
                eout_ref[pl.ds(vb * BV, BV), pl.ds(cb * BC, BC)] = e
            else:
                xvb = xv_ref[pl.ds(vb * BV, BV), :]
                xcb = xc_ref[pl.ds(cb * BC, BC), :]
                mi_blk = jax.lax.dot_general(
                    e, xvb, (((0,), (0,)), ((), ())),
                    preferred_element_type=jnp.float32)
                mo_blk = jnp.dot(e, xcb,
                                 preferred_element_type=jnp.float32)
                mi_ref[pl.ds(cb * BC, BC), :] += mi_blk
                mo_ref[pl.ds(vb * BV, BV), :] += mo_blk
            return carry

        jax.lax.fori_loop(0, (NV // BV) * (NC // BC), body, 0)

    for _ in range(3):
        edge_pass(0, 6, False)
        mi = mi_ref[...]
        mo = mo_ref[...]
        x_v = xv_ref[...]
        x_c = xc_ref[...]
        pv = (jnp.dot(mo, M(5), preferred_element_type=jnp.float32)
              + jnp.dot(x_v, M(6), preferred_element_type=jnp.float32) + V(17))
        pc = (jnp.dot(mi, M(4), preferred_element_type=jnp.float32)
              + jnp.dot(x_c, M(6), preferred_element_type=jnp.float32) + V(17))
        hv = jnp.maximum(_ln(pv, V(18), V(19)), 0.0)
        hc = jnp.maximum(_ln(pc, V(18), V(19)), 0.0)
        hv = lnl(hv, M(7), V(20), V(21), V(22))
        hc = lnl(hc, M(7), V(20), V(21), V(22))
        hv = lnl(hv, M(8), V(23), V(24), V(25))
        hc = lnl(hc, M(8), V(23), V(24), V(25))
        hv = jnp.dot(hv, M(9), preferred_element_type=jnp.float32) + V(26)
        hc = jnp.dot(hc, M(9), preferred_element_type=jnp.float32) + V(26)
        hv = jnp.maximum(_ln(hv, V(27), V(28)), 0.0)
        hc = jnp.maximum(_ln(hc, V(27), V(28)), 0.0)
        xv_ref[...] = x_v + hv
        xc_ref[...] = x_c + hc

    edge_pass(10, 29, True)

    x = jnp.concatenate([xv_ref[...], xc_ref[...]], axis=0)
    h = lnl(x, M(14), V(40), V(41), V(42))
    h = lnl(h, M(15), V(43), V(44), V(45))
    h = lnl(h, M(16), V(46), V(47), V(48))
    a = jax.nn.sigmoid(jnp.sum(h * V(49), axis=-1, keepdims=True)
                       + vecs[50, 0])
    a_ref[...] = a


def _row(v):
    return v.reshape(H)


def kernel(vxd_hits, vxd_trackids, vxd_tracks, cdc_hits, cdc_trackids,
           cdc_tracks, params):
    p = params
    ivh, ich = p["in_vxd_hits"], p["in_cdc_hits"]
    sv = _embed_scatter(vxd_hits, vxd_trackids.astype(jnp.int32),
                        ivh["final"]["W"], ivh["final"]["b"],
                        ivh["final_ln"]["g"], ivh["final_ln"]["beta"],
                        NV, 2000)
    sc = _embed_scatter(cdc_hits, cdc_trackids.astype(jnp.int32),
                        ich["final"]["W"], ich["final"]["b"],
                        ich["final_ln"]["g"], ich["final_ln"]["beta"],
                        NC, 2000)

    en, nn = p["edge_network"], p["node_network"]
    ed, nd = p["edge_decoder"], p["node_decoder"]
    mats = jnp.stack([
        en["layers"][0]["W"][:H], en["layers"][0]["W"][H:],
        en["layers"][1]["W"], en["layers"][2]["W"],
        nn["layers"][0]["W"][:H], nn["layers"][0]["W"][H:2 * H],
        nn["layers"][0]["W"][2 * H:],
        nn["layers"][1]["W"], nn["layers"][2]["W"], nn["final"]["W"],
        ed["layers"][0]["W"][:H], ed["layers"][0]["W"][H:],
        ed["layers"][1]["W"], ed["layers"][2]["W"],
        nd["layers"][0]["W"], nd["layers"][1]["W"], nd["layers"][2]["W"],
    ])

    def ln_rows(mlp, i):
        l = mlp["layers"][i]
        return [l["b"], l["g"], l["beta"]]

    itv, itc = p["in_vxd_tracks"], p["in_cdc_tracks"]
    vec_list = [
        itv["final"]["b"], itv["final_ln"]["g"], itv["final_ln"]["beta"],
        itc["final"]["b"], itc["final_ln"]["g"], itc["final_ln"]["beta"],
    ]
    for mlp in (en,):
        vec_list += ln_rows(mlp, 0) + ln_rows(mlp, 1) + ln_rows(mlp, 2)
        vec_list += [mlp["final"]["W"][:, 0], jnp.full((H,), mlp["final"]["b"][0])]
    vec_list += ln_rows(nn, 0) + ln_rows(nn, 1) + ln_rows(nn, 2)
    vec_list += [nn["final"]["b"], nn["final_ln"]["g"], nn["final_ln"]["beta"]]
    for mlp in (ed, nd):
        vec_list += ln_rows(mlp, 0) + ln_rows(mlp, 1) + ln_rows(mlp, 2)
        vec_list += [mlp["final"]["W"][:, 0], jnp.full((H,), mlp["final"]["b"][0])]
    vecs = jnp.stack([_row(v) for v in vec_list])

    eout, a = pl.pallas_call(
        _graph_kernel,
        out_shape=(jax.ShapeDtypeStruct((NV, NC), jnp.float32),
                   jax.ShapeDtypeStruct((NV + NC, 1), jnp.float32)),
        scratch_shapes=[pltpu.VMEM((NC, H), jnp.float32),
                        pltpu.VMEM((NV, H), jnp.float32),
                        pltpu.VMEM((NV, H), jnp.float32),
                        pltpu.VMEM((NC, H), jnp.float32),
                        pltpu.VMEM((NV, H), jnp.float32),
                        pltpu.VMEM((NC, H), jnp.float32)],
    )(sv, sc, vxd_tracks, cdc_tracks, itv["final"]["W"], itc["final"]["W"],
      mats, vecs)
    return (eout.reshape(NV * NC), a.reshape(NV + NC))


# speedup vs baseline: 27.9217x; 3.4620x over previous
import functools

import jax
import jax.numpy as jnp
from jax.experimental import pallas as pl
from jax.experimental.pallas import tpu as pltpu

H = 32
NV = 512
NC = 1024
EPS = 1e-5
BV = 8
TILES = NV // BV
BH = 2048


def _lnT(x, g, b):
    mu = jnp.mean(x, axis=0, keepdims=True)
    var = jnp.mean((x - mu) ** 2, axis=0, keepdims=True)
    return (x - mu) * jax.lax.rsqrt(var + EPS) * g + b


def _tdot(w, x):
    return jax.lax.dot_general(w, x, (((0,), (0,)), ((), ())),
                               preferred_element_type=jnp.float32)


def _embed_scatter_kernel(hitsT_ref, ids_ref, w_ref, b_ref, g_ref, beta_ref,
                          acc_ref, *, n_tracks):
    i = pl.program_id(0)
    hT = _tdot(w_ref[...], hitsT_ref[...]) + b_ref[...]
    hT = jnp.maximum(_lnT(hT, g_ref[...], beta_ref[...]), 0.0)
    ids = ids_ref[...]
    onehot = (jax.lax.broadcasted_iota(jnp.int32, (BH, n_tracks), 1)
              == ids).astype(jnp.float32)
    contrib = jnp.dot(hT, onehot,
                      preferred_element_type=jnp.float32)

    @pl.when(i == 0)
    def _():
        acc_ref[...] = jnp.zeros_like(acc_ref)

    acc_ref[...] += contrib


def _embed_scatter(hits, ids, w, b, g, beta, n_tracks):
    n, f = hits.shape
    npad = -n % BH
    hitsT = jnp.pad(hits, ((0, npad), (0, 0))).T
    ids2 = jnp.pad(ids, (0, npad),
                   constant_values=n_tracks).reshape(n + npad, 1)
    grid = (n + npad) // BH
    b2, g2, beta2 = b.reshape(H, 1), g.reshape(H, 1), beta.reshape(H, 1)
    return pl.pallas_call(
        functools.partial(_embed_scatter_kernel, n_tracks=n_tracks),
        grid=(grid,),
        in_specs=[
            pl.BlockSpec((f, BH), lambda i: (0, i)),
            pl.BlockSpec((BH, 1), lambda i: (i, 0)),
            pl.BlockSpec((f, H), lambda i: (0, 0)),
            pl.BlockSpec((H, 1), lambda i: (0, 0)),
            pl.BlockSpec((H, 1), lambda i: (0, 0)),
            pl.BlockSpec((H, 1), lambda i: (0, 0)),
        ],
        out_specs=pl.BlockSpec((H, n_tracks), lambda i: (0, 0)),
        out_shape=jax.ShapeDtypeStruct((H, n_tracks), jnp.float32),
    )(hitsT, ids2, w, b2, g2, beta2)


def _eye(n):
    return (jax.lax.broadcasted_iota(jnp.int32, (n, n), 0)
            == jax.lax.broadcasted_iota(jnp.int32, (n, n), 1)
            ).astype(jnp.float32)


def _graph_kernel(svT_ref, scT_ref, tvT_ref, tcT_ref, wtv_ref, wtc_ref,
                  mats_ref, vecsT_ref, eout_ref, a_ref,
                  xvT_ref, xcT_ref, miT_ref, moN_ref,
                  AN_ref, xvN_ref, Brep_ref, xcrep_ref, Rv_ref):
    mats = mats_ref[...]

    def M(k):
        return mats[k]

    def Vc(k):
        return vecsT_ref[:, k:k + 1]

    def lnlT(h, w, b, g, t):
        return jnp.maximum(_lnT(_tdot(w, h) + b, g, t), 0.0)

    Rv_ref[...] = (jax.lax.broadcasted_iota(jnp.int32, (BV, BV * NC), 1)
                   // NC ==
                   jax.lax.broadcasted_iota(jnp.int32, (BV, BV * NC), 0)
                   ).astype(jnp.float32)

    xvT_ref[...] = svT_ref[...] + jnp.maximum(
        _lnT(_tdot(wtv_ref[...], tvT_ref[...]) + Vc(0), Vc(1), Vc(2)), 0.0)
    xcT_ref[...] = scT_ref[...] + jnp.maximum(
        _lnT(_tdot(wtc_ref[...], tcT_ref[...]) + Vc(3), Vc(4), Vc(5)), 0.0)

    def edge_pass(mb, vb_, write_out):
        g1, t1 = Vc(vb_ + 1), Vc(vb_ + 2)
        b2, g2, t2 = Vc(vb_ + 3), Vc(vb_ + 4), Vc(vb_ + 5)
        b3, g3, t3 = Vc(vb_ + 6), Vc(vb_ + 7), Vc(vb_ + 8)
        w4c = Vc(vb_ + 9)
        b4 = vecsT_ref[0:1, vb_ + 10:vb_ + 11]
        AN_ref[...] = _tdot(xvT_ref[...], M(mb))
        BT = _tdot(M(mb + 1), xcT_ref[...]) + Vc(vb_)
        Brep_ref[...] = jnp.concatenate([BT] * BV, axis=1)
        if not write_out:
            miT_ref[...] = jnp.zeros_like(miT_ref)
            moN_ref[...] = jnp.zeros_like(moN_ref)
            xvN_ref[...] = _tdot(xvT_ref[...], _eye(H))
            xcrep_ref[...] = jnp.concatenate([xcT_ref[...]] * BV, axis=1)

        def body(k, carry):
            ATblk = AN_ref[pl.ds(k * BV, BV), :]
            ATrep = _tdot(ATblk, Rv_ref[...])
            pre = ATrep + Brep_ref[...]
            h = jnp.maximum(_lnT(pre, g1, t1), 0.0)
            h = lnlT(h, M(mb + 2), b2, g2, t2)
            h = lnlT(h, M(mb + 3), b3, g3, t3)
            e = jax.nn.sigmoid(jnp.sum(h * w4c, axis=0, keepdims=True)
                               + b4)
            if write_out:
                eout_ref[pl.ds(k, 1), :] = e
            else:
                xvblk = xvN_ref[pl.ds(k * BV, BV), :]
                xvrep = _tdot(xvblk, Rv_ref[...])
                Y2 = xvrep * e
                mi_c = Y2[:, 0:NC]
                for j in range(1, BV):
                    mi_c = mi_c + Y2[:, j * NC:(j + 1) * NC]
                miT_ref[...] += mi_c
                Y = xcrep_ref[...] * e
                mo_blk = jax.lax.dot_general(
                    Rv_ref[...], Y, (((1,), (1,)), ((), ())),
                    preferred_element_type=jnp.float32)
                moN_ref[pl.ds(k * BV, BV), :] += mo_blk
            return carry

        jax.lax.fori_loop(0, TILES, body, 0)

    for _ in range(3):
        edge_pass(0, 6, False)
        miT = miT_ref[...]
        moT = _tdot(moN_ref[...], _eye(NV))
        xvT = xvT_ref[...]
        xcT = xcT_ref[...]
        pv = _tdot(M(5), moT) + _tdot(M(6), xvT) + Vc(17)
        pc = _tdot(M(4), miT) + _tdot(M(6), xcT) + Vc(17)
        hv = jnp.maximum(_lnT(pv, Vc(18), Vc(19)), 0.0)
        hc = jnp.maximum(_lnT(pc, Vc(18), Vc(19)), 0.0)
        hv = lnlT(hv, M(7), Vc(20), Vc(21), Vc(22))
        hc = lnlT(hc, M(7), Vc(20), Vc(21), Vc(22))
        hv = lnlT(hv, M(8), Vc(23), Vc(24), Vc(25))
        hc = lnlT(hc, M(8), Vc(23), Vc(24), Vc(25))
        hv = _tdot(M(9), hv) + Vc(26)
        hc = _tdot(M(9), hc) + Vc(26)
        hv = jnp.maximum(_lnT(hv, Vc(27), Vc(28)), 0.0)
        hc = jnp.maximum(_lnT(hc, Vc(27), Vc(28)), 0.0)
        xvT_ref[...] = xvT + hv
        xcT_ref[...] = xcT + hc

    edge_pass(10, 29, True)

    xT = jnp.concatenate([xvT_ref[...], xcT_ref[...]], axis=1)
    h = lnlT(xT, M(14), Vc(40), Vc(41), Vc(42))
    h = lnlT(h, M(15), Vc(43), Vc(44), Vc(45))
    h = lnlT(h, M(16), Vc(46), Vc(47), Vc(48))
    a_ref[...] = jax.nn.sigmoid(
        jnp.sum(h * Vc(49), axis=0, keepdims=True)
        + vecsT_ref[0:1, 50:51])


def _row(v):
    return v.reshape(H)


def kernel(vxd_hits, vxd_trackids, vxd_tracks, cdc_hits, cdc_trackids,
           cdc_tracks, params):
    p = params
    ivh, ich = p["in_vxd_hits"], p["in_cdc_hits"]
    svT = _embed_scatter(vxd_hits, vxd_trackids.astype(jnp.int32),
                         ivh["final"]["W"], ivh["final"]["b"],
                         ivh["final_ln"]["g"], ivh["final_ln"]["beta"], NV)
    scT = _embed_scatter(cdc_hits, cdc_trackids.astype(jnp.int32),
                         ich["final"]["W"], ich["final"]["b"],
                         ich["final_ln"]["g"], ich["final_ln"]["beta"], NC)

    en, nn = p["edge_network"], p["node_network"]
    ed, nd = p["edge_decoder"], p["node_decoder"]
    mats = jnp.stack([
        en["layers"][0]["W"][:H], en["layers"][0]["W"][H:],
        en["layers"][1]["W"], en["layers"][2]["W"],
        nn["layers"][0]["W"][:H], nn["layers"][0]["W"][H:2 * H],
        nn["layers"][0]["W"][2 * H:],
        nn["layers"][1]["W"], nn["layers"][2]["W"], nn["final"]["W"],
        ed["layers"][0]["W"][:H], ed["layers"][0]["W"][H:],
        ed["layers"][1]["W"], ed["layers"][2]["W"],
        nd["layers"][0]["W"], nd["layers"][1]["W"], nd["layers"][2]["W"],
    ])

    def ln_rows(mlp, i):
        l = mlp["layers"][i]
        return [l["b"], l["g"], l["beta"]]

    itv, itc = p["in_vxd_tracks"], p["in_cdc_tracks"]
    vec_list = [
        itv["final"]["b"], itv["final_ln"]["g"], itv["final_ln"]["beta"],
        itc["final"]["b"], itc["final_ln"]["g"], itc["final_ln"]["beta"],
    ]
    for mlp in (en,):
        vec_list += ln_rows(mlp, 0) + ln_rows(mlp, 1) + ln_rows(mlp, 2)
        vec_list += [mlp["final"]["W"][:, 0], jnp.full((H,), mlp["final"]["b"][0])]
    vec_list += ln_rows(nn, 0) + ln_rows(nn, 1) + ln_rows(nn, 2)
    vec_list += [nn["final"]["b"], nn["final_ln"]["g"], nn["final_ln"]["beta"]]
    for mlp in (ed, nd):
        vec_list += ln_rows(mlp, 0) + ln_rows(mlp, 1) + ln_rows(mlp, 2)
        vec_list += [mlp["final"]["W"][:, 0], jnp.full((H,), mlp["final"]["b"][0])]
    vecsT = jnp.stack([_row(v) for v in vec_list]).T

    eout, a = pl.pallas_call(
        _graph_kernel,
        out_shape=(jax.ShapeDtypeStruct((TILES, BV * NC), jnp.float32),
                   jax.ShapeDtypeStruct((1, NV + NC), jnp.float32)),
        scratch_shapes=[pltpu.VMEM((H, NV), jnp.float32),
                        pltpu.VMEM((H, NC), jnp.float32),
                        pltpu.VMEM((H, NC), jnp.float32),
                        pltpu.VMEM((NV, H), jnp.float32),
                        pltpu.VMEM((NV, H), jnp.float32),
                        pltpu.VMEM((NV, H), jnp.float32),
                        pltpu.VMEM((H, BV * NC), jnp.float32),
                        pltpu.VMEM((H, BV * NC), jnp.float32),
                        pltpu.VMEM((BV, BV * NC), jnp.float32)],
    )(svT, scT, vxd_tracks.T, cdc_tracks.T, itv["final"]["W"],
      itc["final"]["W"], mats, vecsT)
    return (eout.reshape(NV * NC), a.reshape(NV + NC))
